# speedup vs baseline: 3.2649x; 3.2649x over previous
import jax
import jax.numpy as jnp
from jax.experimental import pallas as pl

_BLK_ROWS = 256


def _mask_kernel(mtop_ref, out_ref):
    mtop = mtop_ref[...]
    cols = jax.lax.broadcasted_iota(jnp.int32, out_ref.shape, 1)
    out_ref[...] = cols > mtop


def kernel(m_top, scores):
    B, H, U, L_K = scores.shape
    rows = B * H * U
    grid = rows // _BLK_ROWS
    mt = m_top.reshape(rows, 1).astype(jnp.int32)
    out = pl.pallas_call(
        _mask_kernel,
        grid=(grid,),
        in_specs=[pl.BlockSpec((_BLK_ROWS, 1), lambda i: (i, 0))],
        out_specs=pl.BlockSpec((_BLK_ROWS, L_K), lambda i: (i, 0)),
        out_shape=jax.ShapeDtypeStruct((rows, L_K), jnp.bool_),
    )(mt)
    return out.reshape(B, H, U, L_K)


# speedup vs baseline: 3.3022x; 1.0114x over previous
import jax
import jax.numpy as jnp
from jax.experimental import pallas as pl

_BLK_ROWS = 512


def _mask_kernel(mtop_ref, out_ref):
    mtop = mtop_ref[...]
    cols = jax.lax.broadcasted_iota(jnp.int32, out_ref.shape, 1)
    out_ref[...] = cols > mtop


def kernel(m_top, scores):
    B, H, U, L_K = scores.shape
    rows = B * H * U
    grid = rows // _BLK_ROWS
    mt = m_top.reshape(rows, 1).astype(jnp.int32)
    out = pl.pallas_call(
        _mask_kernel,
        grid=(grid,),
        in_specs=[pl.BlockSpec((_BLK_ROWS, 1), lambda i: (i, 0))],
        out_specs=pl.BlockSpec((_BLK_ROWS, L_K), lambda i: (i, 0)),
        out_shape=jax.ShapeDtypeStruct((rows, L_K), jnp.bool_),
    )(mt)
    return out.reshape(B, H, U, L_K)
